# Initial kernel scaffold; baseline (speedup 1.0000x reference)
import jax
import jax.numpy as jnp
from jax.experimental import pallas as pl


def kernel(input_sequence, embedding_table, f):
    raise NotImplementedError("write your pallas kernel here")


# speedup vs baseline: 14.1521x; 14.1521x over previous
import functools

import jax
import jax.numpy as jnp
from jax import lax
from jax.experimental import pallas as pl
from jax.experimental.pallas import tpu as pltpu
from jax.experimental.pallas import tpu_sc as plsc

B, NF, L, D = 4096, 26, 50, 32
S = B * NF
NW = 32
SEG_PER_W = S // NW
SEG_PER_SUB = 2
IDX_PER_SUB = SEG_PER_SUB * L
N_SUB = 16
SEG_PER_CHUNK = N_SUB * SEG_PER_SUB
CHUNKS = SEG_PER_W // SEG_PER_CHUNK


def _sc_body(idx_hbm, f_hbm, table_hbm, out_hbm, f_v, idx_v, rows_v, out_v, sem):
    wid = lax.axis_index("s") * 2 + lax.axis_index("c")
    pltpu.sync_copy(f_hbm, f_v)
    sub_base0 = wid * (SEG_PER_W // SEG_PER_SUB)
    seg_base0 = wid * SEG_PER_W

    def chunk_body(c, _):
        sub_base = sub_base0 + c * N_SUB
        pltpu.sync_copy(idx_hbm.at[pl.ds(sub_base, N_SUB)], idx_v)
        copies = []
        for j in range(N_SUB):
            copies.append(
                pltpu.async_copy(table_hbm.at[idx_v.at[j]], rows_v.at[j], sem))
        for cp in copies:
            cp.wait()

        def sub_body(j, _):
            acc = [[jnp.zeros((16,), jnp.float32) for _ in range(2)]
                   for _ in range(SEG_PER_SUB)]
            for l in range(L):
                f0 = f_v[l, 0:16]
                f1 = f_v[l, 16:32]
                for s in range(SEG_PER_SUB):
                    r = s * L + l
                    acc[s][0] = acc[s][0] + rows_v[j, r, 0:16] * f0
                    acc[s][1] = acc[s][1] + rows_v[j, r, 16:32] * f1
            for s in range(SEG_PER_SUB):
                out_v[j * SEG_PER_SUB + s, 0:16] = acc[s][0]
                out_v[j * SEG_PER_SUB + s, 16:32] = acc[s][1]
            return 0

        lax.fori_loop(0, N_SUB, sub_body, 0)
        pltpu.sync_copy(
            out_v, out_hbm.at[pl.ds(seg_base0 + c * SEG_PER_CHUNK, SEG_PER_CHUNK)])
        return 0

    lax.fori_loop(0, CHUNKS, chunk_body, 0)


@jax.jit
def _encode(idx2, f50, table):
    mesh = plsc.VectorSubcoreMesh(core_axis_name="c", subcore_axis_name="s")
    run = pl.kernel(
        _sc_body,
        out_type=jax.ShapeDtypeStruct((S, D), jnp.float32),
        mesh=mesh,
        scratch_types=[
            pltpu.VMEM((L, D), jnp.float32),
            pltpu.VMEM((N_SUB, IDX_PER_SUB), jnp.int32),
            pltpu.VMEM((N_SUB, IDX_PER_SUB, D), jnp.float32),
            pltpu.VMEM((SEG_PER_CHUNK, D), jnp.float32),
            pltpu.SemaphoreType.DMA,
        ],
        compiler_params=pltpu.CompilerParams(use_tc_tiling_on_sc=False),
    )
    return run(idx2, f50, table)


def kernel(input_sequence, embedding_table, f):
    idx2 = input_sequence.reshape(S // SEG_PER_SUB, IDX_PER_SUB)
    out = _encode(idx2, f[:L], embedding_table)
    return out.reshape(B, NF, D)


# speedup vs baseline: 17.6255x; 1.2454x over previous
import jax
import jax.numpy as jnp
from jax import lax
from jax.experimental import pallas as pl
from jax.experimental.pallas import tpu as pltpu
from jax.experimental.pallas import tpu_sc as plsc

B, NF, L, D = 4096, 26, 50, 32
S = B * NF
NW = 32
SEG_PER_W = S // NW
SEG_PER_SUB = 2
IDX_PER_SUB = SEG_PER_SUB * L
N_SUB = 16
SEG_PER_CHUNK = N_SUB * SEG_PER_SUB
CHUNKS = SEG_PER_W // SEG_PER_CHUNK
PAIRS = CHUNKS // 2


def _sc_body(idx_hbm, f_hbm, table_hbm, out_hbm,
             f_v, idx_v, rows_v, out_v, gsem0, gsem1, osem0, osem1):
    wid = lax.axis_index("s") * 2 + lax.axis_index("c")
    pltpu.sync_copy(f_hbm, f_v)
    sub_base0 = wid * (SEG_PER_W // SEG_PER_SUB)
    seg_base0 = wid * SEG_PER_W
    gsems = (gsem0, gsem1)
    osems = (osem0, osem1)

    def gather_copies(slot):
        gsem = gsems[slot]
        return [pltpu.make_async_copy(table_hbm.at[idx_v.at[slot].at[j]],
                                      rows_v.at[slot].at[j], gsem)
                for j in range(N_SUB)]

    def stage_fire(c, slot):
        sub_base = sub_base0 + c * N_SUB
        pltpu.sync_copy(idx_hbm.at[pl.ds(sub_base, N_SUB)], idx_v.at[slot])
        for cp in gather_copies(slot):
            cp.start()

    def drain(slot):
        for cp in gather_copies(slot):
            cp.wait()

    def out_copy(c, slot):
        return pltpu.make_async_copy(
            out_v.at[slot],
            out_hbm.at[pl.ds(seg_base0 + c * SEG_PER_CHUNK, SEG_PER_CHUNK)],
            osems[slot])

    def compute(slot):
        def group_body(g, _):
            acc = [[jnp.zeros((16,), jnp.float32) for _ in range(2)]
                   for _ in range(4)]
            for l in range(L):
                f0 = f_v[l, 0:16]
                f1 = f_v[l, 16:32]
                for sj in range(2):
                    for s in range(2):
                        r = s * L + l
                        k = 2 * sj + s
                        acc[k][0] = acc[k][0] + rows_v[slot, 2 * g + sj, r, 0:16] * f0
                        acc[k][1] = acc[k][1] + rows_v[slot, 2 * g + sj, r, 16:32] * f1
            for k in range(4):
                out_v[slot, 4 * g + k, 0:16] = acc[k][0]
                out_v[slot, 4 * g + k, 16:32] = acc[k][1]
            return 0

        lax.fori_loop(0, N_SUB // 2, group_body, 0)

    stage_fire(0, 0)

    def pair_body(p, _):
        c0 = 2 * p
        c1 = c0 + 1
        stage_fire(c1, 1)

        drain(0)

        @pl.when(p > 0)
        def _():
            out_copy(c0, 0).wait()
        compute(0)
        out_copy(c0, 0).start()

        @pl.when(p < PAIRS - 1)
        def _():
            stage_fire(c0 + 2, 0)

        drain(1)

        @pl.when(p > 0)
        def _():
            out_copy(c1, 1).wait()
        compute(1)
        out_copy(c1, 1).start()
        return 0

    lax.fori_loop(0, PAIRS, pair_body, 0)
    out_copy(CHUNKS - 2, 0).wait()
    out_copy(CHUNKS - 1, 1).wait()


@jax.jit
def _encode(idx2, f50, table):
    mesh = plsc.VectorSubcoreMesh(core_axis_name="c", subcore_axis_name="s")
    run = pl.kernel(
        _sc_body,
        out_type=jax.ShapeDtypeStruct((S, D), jnp.float32),
        mesh=mesh,
        scratch_types=[
            pltpu.VMEM((L, D), jnp.float32),
            pltpu.VMEM((2, N_SUB, IDX_PER_SUB), jnp.int32),
            pltpu.VMEM((2, N_SUB, IDX_PER_SUB, D), jnp.float32),
            pltpu.VMEM((2, SEG_PER_CHUNK, D), jnp.float32),
            pltpu.SemaphoreType.DMA,
            pltpu.SemaphoreType.DMA,
            pltpu.SemaphoreType.DMA,
            pltpu.SemaphoreType.DMA,
        ],
        compiler_params=pltpu.CompilerParams(use_tc_tiling_on_sc=False),
    )
    return run(idx2, f50, table)


def kernel(input_sequence, embedding_table, f):
    idx2 = input_sequence.reshape(S // SEG_PER_SUB, IDX_PER_SUB)
    out = _encode(idx2, f[:L], embedding_table)
    return out.reshape(B, NF, D)
